# Initial kernel scaffold; baseline (speedup 1.0000x reference)
import jax
import jax.numpy as jnp
from jax.experimental import pallas as pl


def kernel(obs_flat, seq_lens, num_nodes, nodes, adj_mats, W_rel0, b_rel0, W_root0, W_rel1, b_rel1, W_root1, W_logit, b_logit, W_value, b_value):
    raise NotImplementedError("write your pallas kernel here")


# speedup vs baseline: 73.5562x; 73.5562x over previous
import functools

import jax
import jax.numpy as jnp
from jax import lax
from jax.experimental import pallas as pl
from jax.experimental.pallas import tpu as pltpu

GS = 128
D = 256
H2 = 1024
NOUT = 512


def _phase1_body(nn_ref, adj_ref, nodes_ref, obs_ref, wrel_ref, wroot_ref,
                 brel_ref, g_ref, r_ref):
    b = pl.program_id(0)
    n = nn_ref[b]
    rowi = lax.broadcasted_iota(jnp.int32, (GS, GS), 0)
    coli = lax.broadcasted_iota(jnp.int32, (GS, GS), 1)
    A = (adj_ref[0] != 0).astype(jnp.float32)
    sel = (rowi == n) & (coli == n)
    sel = sel | ((rowi == n) & (coli == n - 1) & (n > 0))
    sel = sel | ((rowi == n - 1) & (coli == n) & (n > 0))
    A = jnp.where(sel, 1.0, A)
    rmask = lax.broadcasted_iota(jnp.int32, (GS, 1), 0) == n
    X = jnp.where(rmask, jnp.broadcast_to(obs_ref[0, 0, :][None, :], (GS, D)),
                  nodes_ref[0])
    agg0 = lax.dot_general(A, X, (((0,), (0,)), ((), ())),
                           preferred_element_type=jnp.float32)
    h1 = agg0 @ wrel_ref[...] + X @ wroot_ref[...] + brel_ref[0, :][None, :]
    h1 = jnp.maximum(h1, 0.0)
    wcol = jnp.where(coli == n, A, 0.0).sum(axis=1)
    g_ref[0, 0, :] = lax.dot_general(wcol[None, :], h1,
                                     (((1,), (0,)), ((), ())),
                                     preferred_element_type=jnp.float32)[0]
    r_ref[0, 0, :] = jnp.where(rmask, h1, 0.0).sum(axis=0)


def _phase2_body(g_ref, r_ref, wrel_ref, wroot_ref, brel_ref, whead_ref,
                 bhead_ref, out_ref):
    h2 = (g_ref[...] @ wrel_ref[...] + r_ref[...] @ wroot_ref[...]
          + brel_ref[...])
    h2 = jnp.maximum(h2, 0.0)
    out_ref[...] = h2 @ whead_ref[...] + bhead_ref[...]


def _scatter_body(src_ref, out_ref):
    t = pl.program_id(0)

    @pl.when(t == 0)
    def _():
        out_ref[0] = src_ref[...]

    @pl.when(t != 0)
    def _():
        out_ref[...] = jnp.zeros(out_ref.shape, out_ref.dtype)


def kernel(obs_flat, seq_lens, num_nodes, nodes, adj_mats, W_rel0, b_rel0,
           W_root0, W_rel1, b_rel1, W_root1, W_logit, b_logit, W_value,
           b_value):
    B = seq_lens.shape[0]
    T = obs_flat.shape[0] // B
    obs0 = obs_flat.reshape(B, T, D)[:, 0, :]
    nn = num_nodes.reshape(B).astype(jnp.int32)

    grid_spec = pltpu.PrefetchScalarGridSpec(
        num_scalar_prefetch=1,
        grid=(B,),
        in_specs=[
            pl.BlockSpec((1, GS, GS), lambda b, nn_: (b, 0, 0)),
            pl.BlockSpec((1, GS, D), lambda b, nn_: (b, 0, 0)),
            pl.BlockSpec((1, 1, D), lambda b, nn_: (b, 0, 0)),
            pl.BlockSpec((D, D), lambda b, nn_: (0, 0)),
            pl.BlockSpec((D, D), lambda b, nn_: (0, 0)),
            pl.BlockSpec((1, D), lambda b, nn_: (0, 0)),
        ],
        out_specs=[
            pl.BlockSpec((1, 1, D), lambda b, nn_: (b, 0, 0)),
            pl.BlockSpec((1, 1, D), lambda b, nn_: (b, 0, 0)),
        ],
    )
    G, R = pl.pallas_call(
        _phase1_body,
        grid_spec=grid_spec,
        out_shape=[
            jax.ShapeDtypeStruct((B, 1, D), jnp.float32),
            jax.ShapeDtypeStruct((B, 1, D), jnp.float32),
        ],
    )(nn, adj_mats, nodes, obs0.reshape(B, 1, D), W_rel0.T, W_root0.T,
      b_rel0.reshape(1, D))
    G = G.reshape(B, D)
    R = R.reshape(B, D)

    w_head = jnp.concatenate(
        [W_logit.T, W_value.T, jnp.zeros((H2, NOUT - 1), jnp.float32)], axis=1)
    b_head = jnp.concatenate(
        [b_logit, b_value, jnp.zeros((NOUT - 1,), jnp.float32)])
    out2 = pl.pallas_call(
        _phase2_body,
        out_shape=jax.ShapeDtypeStruct((B, 2 * NOUT), jnp.float32),
    )(G, R, W_rel1.T, W_root1.T, b_rel1.reshape(1, H2), w_head,
      b_head.reshape(1, 2 * NOUT))

    packed = pl.pallas_call(
        _scatter_body,
        grid=(T, B),
        in_specs=[pl.BlockSpec((1, 1, 2 * NOUT), lambda t, b: (b, 0, 0))],
        out_specs=pl.BlockSpec((1, 1, 1, 2 * NOUT), lambda t, b: (b, t, 0, 0)),
        out_shape=jax.ShapeDtypeStruct((B, T, 1, 2 * NOUT), jnp.float32),
    )(out2.reshape(B, 1, 2 * NOUT))

    logits = packed[:, :, 0, :NOUT].reshape(B * T, NOUT)
    values = packed[:, :, 0, NOUT].reshape(B * T)
    return (logits, values)


# speedup vs baseline: 353.4352x; 4.8050x over previous
import functools

import jax
import jax.numpy as jnp
from jax import lax
from jax.experimental import pallas as pl
from jax.experimental.pallas import tpu as pltpu

GS = 128
D = 256
H2 = 1024
NOUT = 512
GB = 8


def _phase1_body(nn_ref, adj_ref, nodes_ref, obs_ref, wcat_ref, brel_ref,
                 g_ref, r_ref):
    blk = pl.program_id(0)
    rowi = lax.broadcasted_iota(jnp.int32, (GS, GS), 0)
    coli = lax.broadcasted_iota(jnp.int32, (GS, GS), 1)
    rvec = lax.broadcasted_iota(jnp.int32, (GS, 1), 0)
    for i in range(GB):
        n = nn_ref[blk * GB + i]
        A = (adj_ref[i] != 0).astype(jnp.float32)
        sel = (rowi == n) & (coli == n)
        sel = sel | ((rowi == n) & (coli == n - 1) & (n > 0))
        sel = sel | ((rowi == n - 1) & (coli == n) & (n > 0))
        A = jnp.where(sel, 1.0, A)
        rmask = rvec == n
        X = jnp.where(rmask,
                      jnp.broadcast_to(obs_ref[i, 0, :][None, :], (GS, D)),
                      nodes_ref[i])
        agg0 = lax.dot_general(A, X, (((0,), (0,)), ((), ())),
                               preferred_element_type=jnp.float32)
        Z = jnp.concatenate([agg0, X], axis=1)
        h1 = jnp.maximum(Z @ wcat_ref[...] + brel_ref[0, :][None, :], 0.0)
        wcol = jnp.where(coli == n, A, 0.0).sum(axis=1)
        en = (rvec[:, 0] == n).astype(jnp.float32)
        sel2 = jnp.concatenate([wcol[None, :], en[None, :]], axis=0)
        GR = lax.dot_general(sel2, h1, (((1,), (0,)), ((), ())),
                             preferred_element_type=jnp.float32)
        g_ref[i, 0, :] = GR[0]
        r_ref[i, 0, :] = GR[1]


def _phase2_body(g_ref, r_ref, wrel_ref, wroot_ref, brel_ref, whead_ref,
                 bhead_ref, lg_ref, vl_ref):
    B = g_ref.shape[0]
    h2 = (g_ref[...] @ wrel_ref[...] + r_ref[...] @ wroot_ref[...]
          + brel_ref[...])
    h2 = jnp.maximum(h2, 0.0)
    res = h2 @ whead_ref[...] + bhead_ref[...]
    lg_ref[...] = jnp.concatenate(
        [res[:, None, :NOUT], jnp.zeros((B, 3, NOUT), jnp.float32)], axis=1)
    vl_ref[...] = jnp.concatenate(
        [res[:, None, NOUT:NOUT + 128],
         jnp.zeros((B, 3, 128), jnp.float32)], axis=1)


def kernel(obs_flat, seq_lens, num_nodes, nodes, adj_mats, W_rel0, b_rel0,
           W_root0, W_rel1, b_rel1, W_root1, W_logit, b_logit, W_value,
           b_value):
    B = seq_lens.shape[0]
    T = obs_flat.shape[0] // B
    obs0 = obs_flat.reshape(B, T, D)[:, 0, :]
    nn = num_nodes.reshape(B).astype(jnp.int32)
    w_cat0 = jnp.concatenate([W_rel0.T, W_root0.T], axis=0)

    grid_spec = pltpu.PrefetchScalarGridSpec(
        num_scalar_prefetch=1,
        grid=(B // GB,),
        in_specs=[
            pl.BlockSpec((GB, GS, GS), lambda b, nn_: (b, 0, 0)),
            pl.BlockSpec((GB, GS, D), lambda b, nn_: (b, 0, 0)),
            pl.BlockSpec((GB, 1, D), lambda b, nn_: (b, 0, 0)),
            pl.BlockSpec((2 * D, D), lambda b, nn_: (0, 0)),
            pl.BlockSpec((1, D), lambda b, nn_: (0, 0)),
        ],
        out_specs=[
            pl.BlockSpec((GB, 1, D), lambda b, nn_: (b, 0, 0)),
            pl.BlockSpec((GB, 1, D), lambda b, nn_: (b, 0, 0)),
        ],
    )
    G, R = pl.pallas_call(
        _phase1_body,
        grid_spec=grid_spec,
        out_shape=[
            jax.ShapeDtypeStruct((B, 1, D), jnp.float32),
            jax.ShapeDtypeStruct((B, 1, D), jnp.float32),
        ],
    )(nn, adj_mats, nodes, obs0.reshape(B, 1, D), w_cat0,
      b_rel0.reshape(1, D))
    G = G.reshape(B, D)
    R = R.reshape(B, D)

    w_head = jnp.concatenate(
        [W_logit.T, W_value.T, jnp.zeros((H2, 127), jnp.float32)], axis=1)
    b_head = jnp.concatenate(
        [b_logit, b_value, jnp.zeros((127,), jnp.float32)])
    lg, vl = pl.pallas_call(
        _phase2_body,
        out_shape=[
            jax.ShapeDtypeStruct((B, T, NOUT), jnp.float32),
            jax.ShapeDtypeStruct((B, T, 128), jnp.float32),
        ],
    )(G, R, W_rel1.T, W_root1.T, b_rel1.reshape(1, H2), w_head,
      b_head.reshape(1, NOUT + 128))

    logits = lg.reshape(B * T, NOUT)
    values = vl[:, :, 0].reshape(B * T)
    return (logits, values)
